# Initial kernel scaffold; baseline (speedup 1.0000x reference)
import jax
import jax.numpy as jnp
from jax.experimental import pallas as pl


def kernel(w0, b0, w1, b1, w2, b2, mlp_w1, mlp_b1, mlp_w2, mlp_b2):
    raise NotImplementedError("write your pallas kernel here")


# speedup vs baseline: 4.1181x; 4.1181x over previous
import functools

import jax
import jax.numpy as jnp
from jax import lax
from jax.experimental import pallas as pl

def _keys_of(x):
    b = lax.bitcast_convert_type(x, jnp.int32)
    return jnp.where(b >= 0, b, jnp.bitwise_not(jnp.bitwise_and(b, jnp.int32(2147483647))))


def _float_of(k):
    b = jnp.where(k >= 0, k, jnp.bitwise_or(jnp.bitwise_not(k), jnp.int32(-2147483648)))
    return lax.bitcast_convert_type(b, jnp.float32)


def _row_stats(x):
    R, n = x.shape
    inv_n = jnp.float32(1.0 / n)
    mean = jnp.sum(x, axis=1, keepdims=True) * inv_n
    var = jnp.sum((x - mean) ** 2, axis=1, keepdims=True) * jnp.float32(
        1.0 / (n - 1))
    mn = jnp.min(x, axis=1, keepdims=True)
    mx = jnp.max(x, axis=1, keepdims=True)

    key = _keys_of(x)
    ks = (n // 4, n // 2, (3 * n) // 4)

    def body(i, vs):
        shift = jnp.left_shift(jnp.int32(1), jnp.int32(31) - i)
        out = []
        for j in range(3):
            t = vs[j] + shift
            cnt = jnp.sum((key < t[:, None]).astype(jnp.int32), axis=1)
            out.append(jnp.where(cnt <= ks[j], t, vs[j]))
        return tuple(out)

    v0 = jnp.full((R,), -2147483648, dtype=jnp.int32)
    vs = lax.fori_loop(0, 32, body, (v0, v0, v0))

    qs = []
    fracs = (0.75, 0.5, 0.25)
    for j in range(3):
        v = vs[j]
        below = key < v[:, None]
        cfin = jnp.sum(below.astype(jnp.int32), axis=1)
        lo_key = jnp.max(jnp.where(below, key, jnp.int32(-2147483648)), axis=1)
        lo_key = jnp.where(cfin <= ks[j] - 1, v, lo_key)
        hi = _float_of(v)
        lo = _float_of(lo_key)
        f = jnp.float32(fracs[j])
        qs.append(((1.0 - f) * lo + f * hi)[:, None])

    return jnp.concatenate([mean, var, mn, qs[0], qs[1], qs[2], mx], axis=1)


def _fused_kernel(w0, b0, w1, b1, w2, b2, w1t, w2t, bias1, bias2, out):
    feats = []
    for p in (w0, b0, w1, b1, w2, b2):
        feats.append(_row_stats(p[...]))
    feats.append(jnp.zeros((feats[0].shape[0], 6), dtype=jnp.float32))
    f = jnp.concatenate(feats, axis=1)
    h = jnp.dot(f, w1t[...], preferred_element_type=jnp.float32)
    h = jnp.maximum(h + bias1[...], 0.0)
    out[...] = jnp.dot(h, w2t[...],
                       preferred_element_type=jnp.float32) + bias2[...]


def kernel(w0, b0, w1, b1, w2, b2, mlp_w1, mlp_b1, mlp_w2, mlp_b2):
    B = w0.shape[0]
    R = 128
    grid = (B // R,)

    w0f = w0.reshape(B, -1)
    w1f = w1.reshape(B, -1)
    w2f = w2.reshape(B, -1)

    w1t = jnp.pad(mlp_w1, ((0, 0), (0, 6))).T
    w2t = mlp_w2.T
    bias1 = mlp_b1.reshape(1, -1)
    bias2 = mlp_b2.reshape(1, -1)

    H = mlp_w2.shape[0]
    nw = w0f.shape[1]
    nb = b0.shape[-1]

    row_spec_w = pl.BlockSpec((R, nw), lambda i: (i, 0))
    row_spec_b = pl.BlockSpec((R, nb), lambda i: (i, 0))
    full = lambda a: pl.BlockSpec(a.shape, lambda i: tuple(0 for _ in a.shape))

    return pl.pallas_call(
        _fused_kernel,
        grid=grid,
        in_specs=[
            row_spec_w, row_spec_b, row_spec_w, row_spec_b, row_spec_w,
            row_spec_b, full(w1t), full(w2t), full(bias1), full(bias2),
        ],
        out_specs=pl.BlockSpec((R, H), lambda i: (i, 0)),
        out_shape=jax.ShapeDtypeStruct((B, H), jnp.float32),
    )(w0f, b0, w1f, b1, w2f, b2, w1t, w2t, bias1, bias2)
